# Initial kernel scaffold; baseline (speedup 1.0000x reference)
import functools

import jax
import jax.numpy as jnp
from jax import lax
from jax.experimental import pallas as pl
from jax.experimental.pallas import tpu as pltpu
from jax.experimental.pallas import tpu_sc as plsc

_NC = 2
_NS = 16
_NW = _NC * _NS
_CHUNK = 80
_LDEG = 16

_f32 = jnp.float32


def _mesh():
    return plsc.VectorSubcoreMesh(core_axis_name="c", subcore_axis_name="s")


def _sc_embed_deg(node_tokens, dst3, embed_table, zeros_deg, ones_deg):
    n = node_tokens.shape[0]
    d_emb = embed_table.shape[1]
    ec = dst3.shape[1]
    n_chunks = n // _CHUNK
    cpt = -(-n_chunks // _NW)
    rpt = n // _NS

    @functools.partial(
        pl.kernel,
        out_type=[
            jax.ShapeDtypeStruct((n, d_emb), _f32),
            jax.ShapeDtypeStruct((_NC, n, _LDEG), _f32),
        ],
        mesh=_mesh(),
        scratch_types=[
            pltpu.VMEM((_CHUNK,), jnp.int32),
            pltpu.VMEM((_CHUNK, d_emb), _f32),
            pltpu.VMEM((ec, _CHUNK), jnp.int32),
            pltpu.VMEM((_CHUNK, _LDEG), _f32),
            pltpu.VMEM((rpt, _LDEG), _f32),
            pltpu.VMEM_SHARED((n, _LDEG), _f32),
        ],
    )
    def k(tok_hbm, dst_hbm, table_hbm, zdeg_hbm, ones_hbm,
          e_hbm, pdeg_hbm, tok_v, rows_v, dsts_v, ones_v, stage_v, deg_sh):
        cid = lax.axis_index("c")
        sid = lax.axis_index("s")
        wid = cid * _NS + sid

        pltpu.sync_copy(zdeg_hbm, deg_sh.at[pl.ds(sid * rpt, rpt)])
        pltpu.sync_copy(ones_hbm, ones_v)
        pltpu.sync_copy(dst_hbm.at[wid], dsts_v)
        plsc.subcore_barrier()

        @pl.loop(0, ec)
        def _(j):
            pltpu.sync_copy(ones_v, deg_sh.at[dsts_v.at[j]], add=True)

        for t in range(cpt):
            c = wid * cpt + t

            @pl.when(c < n_chunks)
            def _():
                base = c * _CHUNK
                pltpu.sync_copy(tok_hbm.at[pl.ds(base, _CHUNK)], tok_v)
                pltpu.sync_copy(table_hbm.at[tok_v], rows_v)
                pltpu.sync_copy(rows_v, e_hbm.at[pl.ds(base, _CHUNK)])

        plsc.subcore_barrier()
        pltpu.sync_copy(deg_sh.at[pl.ds(sid * rpt, rpt)], stage_v)
        pltpu.sync_copy(stage_v, pdeg_hbm.at[cid, pl.ds(sid * rpt, rpt)])

    return k(node_tokens, dst3, embed_table, zeros_deg, ones_deg)


def _sc_agg(u, src3, dst3, zeros_nd):
    n, d = u.shape
    ec = src3.shape[1]
    rpt = n // _NS
    ob = rpt // 5

    @functools.partial(
        pl.kernel,
        out_type=jax.ShapeDtypeStruct((_NC, n, d), _f32),
        mesh=_mesh(),
        scratch_types=[
            pltpu.VMEM((ec, _CHUNK), jnp.int32),
            pltpu.VMEM((ec, _CHUNK), jnp.int32),
            pltpu.VMEM((_CHUNK, d), _f32),
            pltpu.VMEM((ob, d), _f32),
            pltpu.VMEM_SHARED((n, d), _f32),
        ],
    )
    def k(u_hbm, src_hbm, dst_hbm, z_hbm, p_hbm,
          srcs_v, dsts_v, rows_v, stage_v, acc_sh):
        cid = lax.axis_index("c")
        sid = lax.axis_index("s")
        wid = cid * _NS + sid

        pltpu.sync_copy(z_hbm, acc_sh.at[pl.ds(sid * rpt, rpt)])
        pltpu.sync_copy(src_hbm.at[wid], srcs_v)
        pltpu.sync_copy(dst_hbm.at[wid], dsts_v)
        plsc.subcore_barrier()

        @pl.loop(0, ec)
        def _(j):
            pltpu.sync_copy(u_hbm.at[srcs_v.at[j]], rows_v)
            pltpu.sync_copy(rows_v, acc_sh.at[dsts_v.at[j]], add=True)

        plsc.subcore_barrier()
        for t in range(5):
            r0 = sid * rpt + t * ob
            pltpu.sync_copy(acc_sh.at[pl.ds(r0, ob)], stage_v)
            pltpu.sync_copy(stage_v, p_hbm.at[cid, pl.ds(r0, ob)])

    return k(u, src3, dst3, zeros_nd)


def _dis_from_pdeg(pdeg_blk):
    deg = pdeg_blk[0, :, 0:1] + pdeg_blk[1, :, 0:1] + 1.0
    return jnp.where(deg > 0, lax.rsqrt(jnp.maximum(deg, 1e-12)), 0.0)


def _tc_u1(e, pdeg, w_node, w1, b_node, blk):
    n, d_emb = e.shape
    d = w1.shape[1]
    grid = n // blk

    def body(e_ref, pdeg_ref, wn_ref, w1_ref, bn_ref, u1_ref):
        dis = _dis_from_pdeg(pdeg_ref[...])
        wf = jnp.dot(wn_ref[...], w1_ref[...], preferred_element_type=_f32)
        bf = jnp.dot(bn_ref[...].reshape(1, -1), w1_ref[...],
                     preferred_element_type=_f32)
        u1_ref[...] = dis * (
            jnp.dot(e_ref[...], wf, preferred_element_type=_f32) + bf)

    return pl.pallas_call(
        body,
        grid=(grid,),
        in_specs=[
            pl.BlockSpec((blk, d_emb), lambda i: (i, 0)),
            pl.BlockSpec((_NC, blk, _LDEG), lambda i: (0, i, 0)),
            pl.BlockSpec((d_emb, d), lambda i: (0, 0)),
            pl.BlockSpec((d, d), lambda i: (0, 0)),
            pl.BlockSpec((d,), lambda i: (0,)),
        ],
        out_specs=pl.BlockSpec((blk, d), lambda i: (i, 0)),
        out_shape=jax.ShapeDtypeStruct((n, d), _f32),
    )(e, pdeg, w_node, w1, b_node)


def _tc_layer1_u2(p1, u1, pdeg, b1, w2, blk):
    n, d = u1.shape
    grid = n // blk

    def body(p_ref, u1_ref, pdeg_ref, b1_ref, w2_ref, u2_ref):
        dis = _dis_from_pdeg(pdeg_ref[...])
        s = p_ref[0] + p_ref[1] + u1_ref[...]
        x1 = jnp.maximum(dis * s + b1_ref[...], 0.0)
        u2_ref[...] = dis * jnp.dot(x1, w2_ref[...],
                                    preferred_element_type=_f32)

    return pl.pallas_call(
        body,
        grid=(grid,),
        in_specs=[
            pl.BlockSpec((_NC, blk, d), lambda i: (0, i, 0)),
            pl.BlockSpec((blk, d), lambda i: (i, 0)),
            pl.BlockSpec((_NC, blk, _LDEG), lambda i: (0, i, 0)),
            pl.BlockSpec((d,), lambda i: (0,)),
            pl.BlockSpec((d, d), lambda i: (0, 0)),
        ],
        out_specs=pl.BlockSpec((blk, d), lambda i: (i, 0)),
        out_shape=jax.ShapeDtypeStruct((n, d), _f32),
    )(p1, u1, pdeg, b1, w2)


def _tc_layer2(p2, u2, pdeg, b2, blk):
    n, d = u2.shape
    grid = n // blk

    def body(p_ref, u2_ref, pdeg_ref, b2_ref, o_ref):
        dis = _dis_from_pdeg(pdeg_ref[...])
        s = p_ref[0] + p_ref[1] + u2_ref[...]
        o_ref[...] = dis * s + b2_ref[...]

    return pl.pallas_call(
        body,
        grid=(grid,),
        in_specs=[
            pl.BlockSpec((_NC, blk, d), lambda i: (0, i, 0)),
            pl.BlockSpec((blk, d), lambda i: (i, 0)),
            pl.BlockSpec((_NC, blk, _LDEG), lambda i: (0, i, 0)),
            pl.BlockSpec((d,), lambda i: (0,)),
        ],
        out_specs=pl.BlockSpec((blk, d), lambda i: (i, 0)),
        out_shape=jax.ShapeDtypeStruct((n, d), _f32),
    )(p2, u2, pdeg, b2)


def kernel(node_tokens, edge_index, embed_table, W_node, b_node, W1, b1, W2, b2):
    n = node_tokens.shape[0]
    e_cnt = edge_index.shape[1]
    d = W1.shape[1]

    node_tokens = node_tokens.astype(jnp.int32)
    edge_index = edge_index.astype(jnp.int32)

    ept = e_cnt // _NW
    ec = ept // _CHUNK
    src3 = edge_index[0].reshape(_NW, ec, _CHUNK)
    dst3 = edge_index[1].reshape(_NW, ec, _CHUNK)

    rpt = n // _NS
    zeros_deg = jnp.zeros((rpt, _LDEG), _f32)
    ones_deg = jnp.ones((_CHUNK, _LDEG), _f32)
    zeros_nd = jnp.zeros((rpt, d), _f32)

    e, pdeg = _sc_embed_deg(node_tokens, dst3, embed_table, zeros_deg, ones_deg)

    blk = 2000
    u1 = _tc_u1(e, pdeg, W_node, W1, b_node, blk)
    p1 = _sc_agg(u1, src3, dst3, zeros_nd)
    u2 = _tc_layer1_u2(p1, u1, pdeg, b1, W2, blk)
    p2 = _sc_agg(u2, src3, dst3, zeros_nd)
    return _tc_layer2(p2, u2, pdeg, b2, blk)


# speedup vs baseline: 2.6427x; 2.6427x over previous
import functools

import jax
import jax.numpy as jnp
from jax import lax
from jax.experimental import pallas as pl
from jax.experimental.pallas import tpu as pltpu
from jax.experimental.pallas import tpu_sc as plsc

_NC = 2
_NS = 16
_NW = _NC * _NS
_LDEG = 16

_f32 = jnp.float32


def _mesh():
    return plsc.VectorSubcoreMesh(core_axis_name="c", subcore_axis_name="s")


def _sc_embed(node_tokens, embed_table):
    n = node_tokens.shape[0]
    d_emb = embed_table.shape[1]
    bpw = n // _NW
    tail = n - bpw * _NW

    @functools.partial(
        pl.kernel,
        out_type=jax.ShapeDtypeStruct((n, d_emb), _f32),
        mesh=_mesh(),
        scratch_types=[
            pltpu.VMEM((bpw,), jnp.int32),
            pltpu.VMEM((bpw, d_emb), _f32),
            pltpu.SemaphoreType.DMA,
        ],
    )
    def k(tok_hbm, table_hbm, e_hbm, tok_v, rows_v, sem):
        cid = lax.axis_index("c")
        sid = lax.axis_index("s")
        wid = cid * _NS + sid
        base = wid * bpw
        pltpu.sync_copy(tok_hbm.at[pl.ds(base, bpw)], tok_v)
        pltpu.async_copy(table_hbm.at[tok_v], rows_v, sem).wait()
        pltpu.sync_copy(rows_v, e_hbm.at[pl.ds(base, bpw)])
        if tail:
            t0 = bpw * _NW
            pltpu.sync_copy(tok_hbm.at[pl.ds(t0, tail)], tok_v.at[pl.ds(0, tail)])
            pltpu.async_copy(table_hbm.at[tok_v.at[pl.ds(0, tail)]],
                             rows_v.at[pl.ds(0, tail)], sem).wait()
            pltpu.sync_copy(rows_v.at[pl.ds(0, tail)], e_hbm.at[pl.ds(t0, tail)])

    return k(node_tokens, embed_table)


def _dis_from_deg(deg_blk):
    return jnp.where(deg_blk > 0,
                     lax.rsqrt(jnp.maximum(deg_blk, 1e-12)), 0.0)


def _tc_u1(e, deg2, w_node, w1, b_node, blk):
    n, d_emb = e.shape
    d = w1.shape[1]

    def body(e_ref, deg_ref, wn_ref, w1_ref, bn_ref, u1_ref):
        dis = _dis_from_deg(deg_ref[...][:, 0:1])
        wf = jnp.dot(wn_ref[...], w1_ref[...], preferred_element_type=_f32)
        bf = jnp.dot(bn_ref[...].reshape(1, -1), w1_ref[...],
                     preferred_element_type=_f32)
        u1_ref[...] = dis * (
            jnp.dot(e_ref[...], wf, preferred_element_type=_f32) + bf)

    return pl.pallas_call(
        body,
        grid=(n // blk,),
        in_specs=[
            pl.BlockSpec((blk, d_emb), lambda i: (i, 0)),
            pl.BlockSpec((blk, _LDEG), lambda i: (i, 0)),
            pl.BlockSpec((d_emb, d), lambda i: (0, 0)),
            pl.BlockSpec((d, d), lambda i: (0, 0)),
            pl.BlockSpec((d,), lambda i: (0,)),
        ],
        out_specs=pl.BlockSpec((blk, d), lambda i: (i, 0)),
        out_shape=jax.ShapeDtypeStruct((n, d), _f32),
    )(e, deg2, w_node, w1, b_node)


def _tc_layer1_u2(agg1, u1, deg2, b1, w2, blk):
    n, d = u1.shape

    def body(a_ref, u1_ref, deg_ref, b1_ref, w2_ref, u2_ref):
        dis = _dis_from_deg(deg_ref[...][:, 0:1])
        x1 = jnp.maximum(dis * (a_ref[...] + u1_ref[...]) + b1_ref[...], 0.0)
        u2_ref[...] = dis * jnp.dot(x1, w2_ref[...],
                                    preferred_element_type=_f32)

    return pl.pallas_call(
        body,
        grid=(n // blk,),
        in_specs=[
            pl.BlockSpec((blk, d), lambda i: (i, 0)),
            pl.BlockSpec((blk, d), lambda i: (i, 0)),
            pl.BlockSpec((blk, _LDEG), lambda i: (i, 0)),
            pl.BlockSpec((d,), lambda i: (0,)),
            pl.BlockSpec((d, d), lambda i: (0, 0)),
        ],
        out_specs=pl.BlockSpec((blk, d), lambda i: (i, 0)),
        out_shape=jax.ShapeDtypeStruct((n, d), _f32),
    )(agg1, u1, deg2, b1, w2)


def _tc_layer2(agg2, u2, deg2, b2, blk):
    n, d = u2.shape

    def body(a_ref, u2_ref, deg_ref, b2_ref, o_ref):
        dis = _dis_from_deg(deg_ref[...][:, 0:1])
        o_ref[...] = dis * (a_ref[...] + u2_ref[...]) + b2_ref[...]

    return pl.pallas_call(
        body,
        grid=(n // blk,),
        in_specs=[
            pl.BlockSpec((blk, d), lambda i: (i, 0)),
            pl.BlockSpec((blk, d), lambda i: (i, 0)),
            pl.BlockSpec((blk, _LDEG), lambda i: (i, 0)),
            pl.BlockSpec((d,), lambda i: (0,)),
        ],
        out_specs=pl.BlockSpec((blk, d), lambda i: (i, 0)),
        out_shape=jax.ShapeDtypeStruct((n, d), _f32),
    )(agg2, u2, deg2, b2)


def kernel(node_tokens, edge_index, embed_table, W_node, b_node, W1, b1, W2, b2):
    n = node_tokens.shape[0]

    node_tokens = node_tokens.astype(jnp.int32)
    edge_index = edge_index.astype(jnp.int32)
    src = edge_index[0]
    dst = edge_index[1]

    e = _sc_embed(node_tokens, embed_table)

    deg = jnp.zeros((n,), _f32).at[dst].add(1.0) + 1.0
    deg2 = jnp.broadcast_to(deg[:, None], (n, _LDEG))

    blk = 2000
    u1 = _tc_u1(e, deg2, W_node, W1, b_node, blk)
    agg1 = jnp.zeros_like(u1).at[dst].add(jnp.take(u1, src, axis=0))
    u2 = _tc_layer1_u2(agg1, u1, deg2, b1, W2, blk)
    agg2 = jnp.zeros_like(u2).at[dst].add(jnp.take(u2, src, axis=0))
    return _tc_layer2(agg2, u2, deg2, b2, blk)


# speedup vs baseline: 2.8499x; 1.0784x over previous
import dataclasses
import functools

import jax
import jax.numpy as jnp
from jax import lax
from jax.experimental import pallas as pl
from jax.experimental.pallas import tpu as pltpu
from jax.experimental.pallas import tpu_sc as plsc

_NC = 2
_NS = 16
_NW = _NC * _NS
_LDEG = 16

_f32 = jnp.float32


def _mesh():
    return plsc.VectorSubcoreMesh(core_axis_name="c", subcore_axis_name="s")


def _sc_embed(node_tokens, embed_table):
    n = node_tokens.shape[0]
    d_emb = embed_table.shape[1]
    bpw = n // _NW
    tail = n - bpw * _NW

    @functools.partial(
        pl.kernel,
        out_type=jax.ShapeDtypeStruct((n, d_emb), _f32),
        mesh=_mesh(),
        scratch_types=[
            pltpu.VMEM((bpw,), jnp.int32),
            pltpu.VMEM((bpw, d_emb), _f32),
            pltpu.SemaphoreType.DMA,
        ],
    )
    def k(tok_hbm, table_hbm, e_hbm, tok_v, rows_v, sem):
        cid = lax.axis_index("c")
        sid = lax.axis_index("s")
        wid = cid * _NS + sid
        base = wid * bpw
        pltpu.sync_copy(tok_hbm.at[pl.ds(base, bpw)], tok_v)
        pltpu.async_copy(table_hbm.at[tok_v], rows_v, sem).wait()
        pltpu.sync_copy(rows_v, e_hbm.at[pl.ds(base, bpw)])
        if tail:
            t0 = bpw * _NW
            pltpu.sync_copy(tok_hbm.at[pl.ds(t0, tail)], tok_v.at[pl.ds(0, tail)])
            pltpu.async_copy(table_hbm.at[tok_v.at[pl.ds(0, tail)]],
                             rows_v.at[pl.ds(0, tail)], sem).wait()
            pltpu.sync_copy(rows_v.at[pl.ds(0, tail)], e_hbm.at[pl.ds(t0, tail)])

    return k(node_tokens, embed_table)


def _sc_deg(dst3, n):
    nw, it, _ = dst3.shape

    cp = pltpu.CompilerParams()
    if "needs_layout_passes" in pltpu.CompilerParams.__dataclass_fields__:
        cp = dataclasses.replace(cp, needs_layout_passes=False)

    @functools.partial(
        pl.kernel,
        out_type=jax.ShapeDtypeStruct((_NW, n), _f32),
        mesh=_mesh(),
        compiler_params=cp,
        scratch_types=[
            pltpu.VMEM((it, 16), jnp.int32),
            pltpu.VMEM((n,), _f32),
        ],
    )
    def k(dst_hbm, p_hbm, dsts_v, hist_v):
        cid = lax.axis_index("c")
        sid = lax.axis_index("s")
        wid = cid * _NS + sid

        @pl.loop(0, n // 16)
        def _(r):
            hist_v[pl.ds(r * 16, 16)] = jnp.zeros((16,), _f32)

        pltpu.sync_copy(dst_hbm.at[wid], dsts_v)

        @pl.loop(0, it)
        def _(j):
            plsc.addupdate_scatter(hist_v, [dsts_v[j, :]],
                                   jnp.ones((16,), _f32))

        pltpu.sync_copy(hist_v, p_hbm.at[wid])

    return k(dst3)


def _dis_from_deg(deg_blk):
    return jnp.where(deg_blk > 0,
                     lax.rsqrt(jnp.maximum(deg_blk, 1e-12)), 0.0)


def _tc_u1(e, deg2, w_node, w1, b_node, blk):
    n, d_emb = e.shape
    d = w1.shape[1]

    def body(e_ref, deg_ref, wn_ref, w1_ref, bn_ref, u1_ref):
        dis = _dis_from_deg(deg_ref[...][:, 0:1])
        wf = jnp.dot(wn_ref[...], w1_ref[...], preferred_element_type=_f32)
        bf = jnp.dot(bn_ref[...].reshape(1, -1), w1_ref[...],
                     preferred_element_type=_f32)
        u1_ref[...] = dis * (
            jnp.dot(e_ref[...], wf, preferred_element_type=_f32) + bf)

    return pl.pallas_call(
        body,
        grid=(n // blk,),
        in_specs=[
            pl.BlockSpec((blk, d_emb), lambda i: (i, 0)),
            pl.BlockSpec((blk, _LDEG), lambda i: (i, 0)),
            pl.BlockSpec((d_emb, d), lambda i: (0, 0)),
            pl.BlockSpec((d, d), lambda i: (0, 0)),
            pl.BlockSpec((d,), lambda i: (0,)),
        ],
        out_specs=pl.BlockSpec((blk, d), lambda i: (i, 0)),
        out_shape=jax.ShapeDtypeStruct((n, d), _f32),
    )(e, deg2, w_node, w1, b_node)


def _tc_layer1_u2(agg1, u1, deg2, b1, w2, blk):
    n, d = u1.shape

    def body(a_ref, u1_ref, deg_ref, b1_ref, w2_ref, u2_ref):
        dis = _dis_from_deg(deg_ref[...][:, 0:1])
        x1 = jnp.maximum(dis * (a_ref[...] + u1_ref[...]) + b1_ref[...], 0.0)
        u2_ref[...] = dis * jnp.dot(x1, w2_ref[...],
                                    preferred_element_type=_f32)

    return pl.pallas_call(
        body,
        grid=(n // blk,),
        in_specs=[
            pl.BlockSpec((blk, d), lambda i: (i, 0)),
            pl.BlockSpec((blk, d), lambda i: (i, 0)),
            pl.BlockSpec((blk, _LDEG), lambda i: (i, 0)),
            pl.BlockSpec((d,), lambda i: (0,)),
            pl.BlockSpec((d, d), lambda i: (0, 0)),
        ],
        out_specs=pl.BlockSpec((blk, d), lambda i: (i, 0)),
        out_shape=jax.ShapeDtypeStruct((n, d), _f32),
    )(agg1, u1, deg2, b1, w2)


def _tc_layer2(agg2, u2, deg2, b2, blk):
    n, d = u2.shape

    def body(a_ref, u2_ref, deg_ref, b2_ref, o_ref):
        dis = _dis_from_deg(deg_ref[...][:, 0:1])
        o_ref[...] = dis * (a_ref[...] + u2_ref[...]) + b2_ref[...]

    return pl.pallas_call(
        body,
        grid=(n // blk,),
        in_specs=[
            pl.BlockSpec((blk, d), lambda i: (i, 0)),
            pl.BlockSpec((blk, d), lambda i: (i, 0)),
            pl.BlockSpec((blk, _LDEG), lambda i: (i, 0)),
            pl.BlockSpec((d,), lambda i: (0,)),
        ],
        out_specs=pl.BlockSpec((blk, d), lambda i: (i, 0)),
        out_shape=jax.ShapeDtypeStruct((n, d), _f32),
    )(agg2, u2, deg2, b2)


def kernel(node_tokens, edge_index, embed_table, W_node, b_node, W1, b1, W2, b2):
    n = node_tokens.shape[0]

    node_tokens = node_tokens.astype(jnp.int32)
    edge_index = edge_index.astype(jnp.int32)
    src = edge_index[0]
    dst = edge_index[1]

    e = _sc_embed(node_tokens, embed_table)

    ept = edge_index.shape[1] // _NW
    pdeg = _sc_deg(dst.reshape(_NW, ept // 16, 16), n)
    deg = jnp.sum(pdeg, axis=0) + 1.0
    deg2 = jnp.broadcast_to(deg[:, None], (n, _LDEG))

    blk = 2000
    u1 = _tc_u1(e, deg2, W_node, W1, b_node, blk)
    agg1 = jnp.zeros_like(u1).at[dst].add(jnp.take(u1, src, axis=0))
    u2 = _tc_layer1_u2(agg1, u1, deg2, b1, W2, blk)
    agg2 = jnp.zeros_like(u2).at[dst].add(jnp.take(u2, src, axis=0))
    return _tc_layer2(agg2, u2, deg2, b2, blk)
